# baseline (speedup 1.0000x reference)
import jax
import jax.numpy as jnp
from jax import lax
from jax.experimental import pallas as pl
from jax.experimental.pallas import tpu as pltpu
from jax.experimental.pallas import tpu_sc as plsc

BINS = 256
BATCH = 16384


_G = 16
_R = 2 * BATCH // _G


_B = BATCH // _G


_H = 4
_BH = _B // _H


def _plane_copy(s_hbm, buf, sems, step, slot):
    base = step * _B
    return tuple(
        pltpu.make_async_copy(
            s_hbm.at[pl.ds(base + h * _BH, _BH), p],
            buf.at[slot, p, pl.ds(h * _BH, _BH)],
            sems.at[slot, p * _H + h])
        for p in (0, 1) for h in range(_H)
    )


def _tc_am_body(s_hbm, out_ref, buf, sems):
    i = pl.program_id(0)

    @pl.when(i == 0)
    def _prime():
        for c in _plane_copy(s_hbm, buf, sems, 0, 0):
            c.start()

    @pl.when(i + 1 < _G)
    def _prefetch():
        for c in _plane_copy(s_hbm, buf, sems, i + 1, (i + 1) % 2):
            c.start()

    cur = i % 2
    for c in _plane_copy(s_hbm, buf, sems, i, cur):
        c.wait()

    def first_argmax(m):
        mx = jnp.max(m, axis=1, keepdims=True)
        io = lax.broadcasted_iota(jnp.int32, m.shape, 1)
        return jnp.min(jnp.where(m == mx, io, BINS), axis=1)

    xi = first_argmax(buf[cur, 0])
    yi = first_argmax(buf[cur, 1])
    out_ref[0, 0, :] = xi * BINS + yi


def _tc_rowidx(s):
    return pl.pallas_call(
        _tc_am_body,
        out_shape=jax.ShapeDtypeStruct((_G, 1, _B), jnp.int32),
        grid=(_G,),
        in_specs=[pl.BlockSpec(memory_space=pl.ANY)],
        out_specs=pl.BlockSpec((1, 1, _B), lambda i: (i, 0, 0)),
        scratch_shapes=[
            pltpu.VMEM((2, 2, _B, BINS), jnp.float32),
            pltpu.SemaphoreType.DMA((2, 2 * _H)),
        ],
    )(s)


_NW = 32
_CH = 128
_NCH = BATCH // _NW // _CH
_L = 16


def _sc_gather_body(ridx_hbm, a_hbm, tbl_hbm, out_hbm,
                    ridx_v, a_v, rows_v, out_v, sem0, sem1):
    cid = lax.axis_index("c")
    sid = lax.axis_index("s")
    wid = sid * 2 + cid
    row0 = wid * _NCH
    pltpu.sync_copy(ridx_hbm.at[pl.ds(row0, _NCH)], ridx_v)
    pltpu.sync_copy(a_hbm.at[pl.ds(row0, _NCH)], a_v)

    lane = lax.iota(jnp.int32, _L)
    sems = (sem0, sem1)
    cp = pltpu.async_copy(tbl_hbm.at[ridx_v.at[0]], rows_v.at[0], sems[0])
    for j in range(_NCH):
        if j + 1 < _NCH:
            nxt = pltpu.async_copy(
                tbl_hbm.at[ridx_v.at[j + 1]], rows_v.at[(j + 1) & 1],
                sems[(j + 1) & 1])
        cp.wait()
        buf = rows_v.at[j & 1]
        for k in range(_CH // _L):
            rl = lane + (k * _L)
            av = a_v[j, pl.ds(k * _L, _L)]
            out_v[j, pl.ds(k * _L, _L)] = plsc.load_gather(buf, [rl, av])
        if j + 1 < _NCH:
            cp = nxt
    pltpu.sync_copy(out_v, out_hbm.at[pl.ds(row0, _NCH)])


def _sc_gather(ridx2, a2, tbl2):
    f = pl.kernel(
        _sc_gather_body,
        out_type=jax.ShapeDtypeStruct((BATCH // _CH, _CH), jnp.float32),
        mesh=plsc.VectorSubcoreMesh(core_axis_name="c", subcore_axis_name="s"),
        scratch_types=[
            pltpu.VMEM((_NCH, _CH), jnp.int32),
            pltpu.VMEM((_NCH, _CH), jnp.int32),
            pltpu.VMEM((2, _CH, BINS), jnp.float32),
            pltpu.VMEM((_NCH, _CH), jnp.float32),
            pltpu.SemaphoreType.DMA,
            pltpu.SemaphoreType.DMA,
        ],
        compiler_params=pltpu.CompilerParams(
            use_tc_tiling_on_sc=True, needs_layout_passes=False),
    )
    return f(ridx2, a2, tbl2)


def kernel(s, a, table):
    ridx = _tc_rowidx(s)
    out = _sc_gather(
        ridx.reshape(BATCH // _CH, _CH),
        a.astype(jnp.int32).reshape(BATCH // _CH, _CH),
        table.reshape(BINS * BINS, BINS),
    )
    return out.reshape(BATCH)


# speedup vs baseline: 1.0576x; 1.0576x over previous
import jax
import jax.numpy as jnp
from jax import lax
from jax.experimental import pallas as pl
from jax.experimental.pallas import tpu as pltpu
from jax.experimental.pallas import tpu_sc as plsc

BINS = 256
BATCH = 16384


_G = 16
_R = 2 * BATCH // _G


_B = BATCH // _G


_H = 4
_BH = _B // _H


def _plane_copy(s_hbm, buf, sems, step, slot):
    base = step * _B
    return tuple(
        pltpu.make_async_copy(
            s_hbm.at[pl.ds(base + h * _BH, _BH), p],
            buf.at[slot, p, pl.ds(h * _BH, _BH)],
            sems.at[slot, p * _H + h])
        for p in (0, 1) for h in range(_H)
    )


def _tc_am_body(s_hbm, out_ref, buf, sems):
    i = pl.program_id(0)

    @pl.when(i == 0)
    def _prime():
        for c in _plane_copy(s_hbm, buf, sems, 0, 0):
            c.start()

    @pl.when(i + 1 < _G)
    def _prefetch():
        for c in _plane_copy(s_hbm, buf, sems, i + 1, (i + 1) % 2):
            c.start()

    cur = i % 2
    for c in _plane_copy(s_hbm, buf, sems, i, cur):
        c.wait()

    def first_argmax(m):
        mx = jnp.max(m, axis=1, keepdims=True)
        io = lax.broadcasted_iota(jnp.int32, m.shape, 1)
        return jnp.min(jnp.where(m == mx, io, BINS), axis=1)

    xi = first_argmax(buf[cur, 0])
    yi = first_argmax(buf[cur, 1])
    out_ref[...] = (xi * BINS + yi).reshape(_B // 128, 128)


def _tc_rowidx(s):
    return pl.pallas_call(
        _tc_am_body,
        out_shape=jax.ShapeDtypeStruct((BATCH // 128, 128), jnp.int32),
        grid=(_G,),
        in_specs=[pl.BlockSpec(memory_space=pl.ANY)],
        out_specs=pl.BlockSpec((_B // 128, 128), lambda i: (i, 0)),
        scratch_shapes=[
            pltpu.VMEM((2, 2, _B, BINS), jnp.float32),
            pltpu.SemaphoreType.DMA((2, 2 * _H)),
        ],
    )(s)


_NW = 32
_CH = 128
_NCH = BATCH // _NW // _CH
_L = 16


def _sc_gather_body(ridx_hbm, a_hbm, tbl_hbm, out_hbm,
                    ridx_v, a_v, rows_v, out_v, sem0, sem1):
    cid = lax.axis_index("c")
    sid = lax.axis_index("s")
    wid = sid * 2 + cid
    row0 = wid * _NCH
    n = _NCH * _CH
    pltpu.sync_copy(ridx_hbm.at[pl.ds(row0, _NCH)], ridx_v)
    pltpu.sync_copy(a_hbm.at[pl.ds(wid * n, n)], a_v)

    lane = lax.iota(jnp.int32, _L)
    sems = (sem0, sem1)
    cp = pltpu.async_copy(tbl_hbm.at[ridx_v.at[0]], rows_v.at[0], sems[0])
    for j in range(_NCH):
        if j + 1 < _NCH:
            nxt = pltpu.async_copy(
                tbl_hbm.at[ridx_v.at[j + 1]], rows_v.at[(j + 1) & 1],
                sems[(j + 1) & 1])
        cp.wait()
        buf = rows_v.at[j & 1]
        for k in range(_CH // _L):
            rl = lane + (k * _L)
            av = a_v[pl.ds(j * _CH + k * _L, _L)]
            out_v[pl.ds(j * _CH + k * _L, _L)] = plsc.load_gather(buf, [rl, av])
        if j + 1 < _NCH:
            cp = nxt
    pltpu.sync_copy(out_v, out_hbm.at[pl.ds(wid * n, n)])


def _sc_gather(ridx2, a1, tbl2):
    f = pl.kernel(
        _sc_gather_body,
        out_type=jax.ShapeDtypeStruct((BATCH,), jnp.float32),
        mesh=plsc.VectorSubcoreMesh(core_axis_name="c", subcore_axis_name="s"),
        scratch_types=[
            pltpu.VMEM((_NCH, _CH), jnp.int32),
            pltpu.VMEM((_NCH * _CH,), jnp.int32),
            pltpu.VMEM((2, _CH, BINS), jnp.float32),
            pltpu.VMEM((_NCH * _CH,), jnp.float32),
            pltpu.SemaphoreType.DMA,
            pltpu.SemaphoreType.DMA,
        ],
        compiler_params=pltpu.CompilerParams(
            use_tc_tiling_on_sc=True, needs_layout_passes=False),
    )
    return f(ridx2, a1, tbl2)


def kernel(s, a, table):
    ridx = _tc_rowidx(s)
    return _sc_gather(ridx, a.astype(jnp.int32),
                      table.reshape(BINS * BINS, BINS))


# speedup vs baseline: 1.0616x; 1.0038x over previous
import jax
import jax.numpy as jnp
from jax import lax
from jax.experimental import pallas as pl
from jax.experimental.pallas import tpu as pltpu
from jax.experimental.pallas import tpu_sc as plsc

BINS = 256
BATCH = 16384


_G = 16
_R = 2 * BATCH // _G


_B = BATCH // _G


_H = 4
_BH = _B // _H


def _plane_copy(s_hbm, buf, sems, step, slot):
    base = step * _B
    return tuple(
        pltpu.make_async_copy(
            s_hbm.at[pl.ds(base + h * _BH, _BH), p],
            buf.at[slot, p, pl.ds(h * _BH, _BH)],
            sems.at[slot, p * _H + h])
        for p in (0, 1) for h in range(_H)
    )


def _tc_am_body(s_hbm, out_ref, buf, sems):
    i = pl.program_id(0)

    @pl.when(i == 0)
    def _prime():
        for c in _plane_copy(s_hbm, buf, sems, 0, 0):
            c.start()

    @pl.when(i + 1 < _G)
    def _prefetch():
        for c in _plane_copy(s_hbm, buf, sems, i + 1, (i + 1) % 2):
            c.start()

    cur = i % 2
    for c in _plane_copy(s_hbm, buf, sems, i, cur):
        c.wait()

    def first_argmax(m):
        mx = jnp.max(m, axis=1, keepdims=True)
        io = lax.broadcasted_iota(jnp.int32, m.shape, 1)
        return jnp.min(jnp.where(m == mx, io, BINS), axis=1)

    xi = first_argmax(buf[cur, 0])
    yi = first_argmax(buf[cur, 1])
    out_ref[...] = (xi * BINS + yi).reshape(_B // 128, 128)


def _tc_rowidx(s):
    return pl.pallas_call(
        _tc_am_body,
        out_shape=jax.ShapeDtypeStruct((BATCH // 128, 128), jnp.int32),
        grid=(_G,),
        in_specs=[pl.BlockSpec(memory_space=pl.ANY)],
        out_specs=pl.BlockSpec((_B // 128, 128), lambda i: (i, 0)),
        scratch_shapes=[
            pltpu.VMEM((2, 2, _B, BINS), jnp.float32),
            pltpu.SemaphoreType.DMA((2, 2 * _H)),
        ],
        compiler_params=pltpu.CompilerParams(skip_device_barrier=True),
    )(s)


_NW = 32
_CH = 128
_NCH = BATCH // _NW // _CH
_L = 16


def _sc_gather_body(ridx_hbm, a_hbm, tbl_hbm, out_hbm,
                    ridx_v, a_v, rows_v, out_v, sem0, sem1):
    cid = lax.axis_index("c")
    sid = lax.axis_index("s")
    wid = sid * 2 + cid
    row0 = wid * _NCH
    n = _NCH * _CH
    pltpu.sync_copy(ridx_hbm.at[pl.ds(row0, _NCH)], ridx_v)
    pltpu.sync_copy(a_hbm.at[pl.ds(wid * n, n)], a_v)

    lane = lax.iota(jnp.int32, _L)
    sems = (sem0, sem1)
    cp = pltpu.async_copy(tbl_hbm.at[ridx_v.at[0]], rows_v.at[0], sems[0])
    for j in range(_NCH):
        if j + 1 < _NCH:
            nxt = pltpu.async_copy(
                tbl_hbm.at[ridx_v.at[j + 1]], rows_v.at[(j + 1) & 1],
                sems[(j + 1) & 1])
        cp.wait()
        buf = rows_v.at[j & 1]
        for k in range(_CH // _L):
            rl = lane + (k * _L)
            av = a_v[pl.ds(j * _CH + k * _L, _L)]
            out_v[pl.ds(j * _CH + k * _L, _L)] = plsc.load_gather(buf, [rl, av])
        if j + 1 < _NCH:
            cp = nxt
    pltpu.sync_copy(out_v, out_hbm.at[pl.ds(wid * n, n)])


def _sc_gather(ridx2, a1, tbl2):
    f = pl.kernel(
        _sc_gather_body,
        out_type=jax.ShapeDtypeStruct((BATCH,), jnp.float32),
        mesh=plsc.VectorSubcoreMesh(core_axis_name="c", subcore_axis_name="s"),
        scratch_types=[
            pltpu.VMEM((_NCH, _CH), jnp.int32),
            pltpu.VMEM((_NCH * _CH,), jnp.int32),
            pltpu.VMEM((2, _CH, BINS), jnp.float32),
            pltpu.VMEM((_NCH * _CH,), jnp.float32),
            pltpu.SemaphoreType.DMA,
            pltpu.SemaphoreType.DMA,
        ],
        compiler_params=pltpu.CompilerParams(
            use_tc_tiling_on_sc=True, needs_layout_passes=False,
            skip_device_barrier=True),
    )
    return f(ridx2, a1, tbl2)


def kernel(s, a, table):
    ridx = _tc_rowidx(s)
    return _sc_gather(ridx, a.astype(jnp.int32),
                      table.reshape(BINS * BINS, BINS))
